# Initial kernel scaffold; baseline (speedup 1.0000x reference)
import jax
import jax.numpy as jnp
from jax import lax
from jax.experimental import pallas as pl
from jax.experimental.pallas import tpu as pltpu
from jax.experimental.pallas import tpu_sc as plsc

N = 10000
E = 320000
D = 128
XE = 3
H = 128
ED = 4

NC = 2
NS = 16
L = 16

EPC = E // NC
EW = E // (NC * NS)
C = 400
NCHUNK = EW // C
ZR = 25

_SLOPE = 0.2

_VMEM_SPEC = pl.BlockSpec(memory_space=pltpu.MemorySpace.VMEM)
_SMEM_SPEC = pl.BlockSpec(memory_space=pltpu.MemorySpace.SMEM)


def _lrelu(x):
  return jnp.where(x >= 0, x, _SLOPE * x)


def _prep_body(x_ref, xe_ref, ewT_ref, w1a_ref, w1b_ref, as1_ref, ad1_ref,
               we1_ref, ae1_ref, we2_ref, ae2_ref,
               h1_ref, ss1_ref, sd1_ref, e1_ref, e2_ref,
               g1v_ref, c1_ref, c2_ref, m2_ref):
  x = x_ref[...]
  xe = xe_ref[...]
  h1 = jnp.dot(x, w1a_ref[...], preferred_element_type=jnp.float32)
  h1 = h1 + jnp.dot(xe, w1b_ref[...], preferred_element_type=jnp.float32)
  h1_ref[...] = h1
  ss1 = jnp.sum(h1 * as1_ref[...][None, :], axis=1, keepdims=True)
  sd1 = jnp.sum(h1 * ad1_ref[...][None, :], axis=1, keepdims=True)
  ss1_ref[...] = ss1
  sd1_ref[...] = sd1
  wvec1 = jnp.sum(we1_ref[...] * ae1_ref[...][:, None], axis=0)
  wvec2 = jnp.sum(we2_ref[...] * ae2_ref[...][:, None], axis=0)
  ewT = ewT_ref[...]
  e1 = jnp.sum(ewT * wvec1[:, None], axis=0)
  e2 = jnp.sum(ewT * wvec2[:, None], axis=0)
  e1_ref[...] = e1
  e2_ref[...] = e2
  c1 = jnp.mean(e1)
  c2 = jnp.mean(e2)
  m1 = jnp.maximum(jnp.max(e1), c1)
  m2 = jnp.maximum(jnp.max(e2), c2)
  g1 = _lrelu(jnp.max(ss1) + jnp.max(sd1) + m1)
  g1v_ref[...] = jnp.full((L,), g1, jnp.float32)
  c1_ref[0, 0] = c1
  c2_ref[0, 0] = c2
  m2_ref[0, 0] = m2


def _combine(acc_ref, den_ref, h_ref, ss_ref, sd_ref, cc, gg, b_ref):
  exl = jnp.exp(_lrelu(ss_ref[...] + sd_ref[...] + cc) - gg)
  den = jnp.sum(den_ref[...], axis=(0, 1))[:, None] + exl + 1e-16
  num = acc_ref[0] + acc_ref[1] + exl * h_ref[...]
  return jax.nn.relu(num / den + b_ref[...][None, :])


def _mid_body(acc_ref, den_ref, h1_ref, ss1_ref, sd1_ref, c1_ref, g1_ref,
              m2_ref, b1_ref, w2_ref, as2_ref, ad2_ref,
              h2_ref, ss2_ref, sd2_ref, g2v_ref, g2_ref):
  x2 = _combine(acc_ref, den_ref, h1_ref, ss1_ref, sd1_ref,
                c1_ref[0, 0], g1_ref[0, 0], b1_ref)
  h2 = jnp.dot(x2, w2_ref[...], preferred_element_type=jnp.float32)
  h2_ref[...] = h2
  ss2 = jnp.sum(h2 * as2_ref[...][None, :], axis=1, keepdims=True)
  sd2 = jnp.sum(h2 * ad2_ref[...][None, :], axis=1, keepdims=True)
  ss2_ref[...] = ss2
  sd2_ref[...] = sd2
  g2 = _lrelu(jnp.max(ss2) + jnp.max(sd2) + m2_ref[0, 0])
  g2v_ref[...] = jnp.full((L,), g2, jnp.float32)
  g2_ref[0, 0] = g2


def _final_body(acc_ref, den_ref, h2_ref, ss2_ref, sd2_ref, c2_ref, g2_ref,
                b2_ref, xe_ref, wla_ref, wlb_ref, bl_ref, out_ref):
  x3 = _combine(acc_ref, den_ref, h2_ref, ss2_ref, sd2_ref,
                c2_ref[0, 0], g2_ref[0, 0], b2_ref)
  z = jnp.dot(x3, wla_ref[...], preferred_element_type=jnp.float32)
  z = z + jnp.dot(xe_ref[...], wlb_ref[...], preferred_element_type=jnp.float32)
  out_ref[...] = jax.nn.relu(z + bl_ref[...][None, :])


def _edge_body(src_h, dst_h, e_h, ss_h, sd_h, h_h, g_h,
               acc_o, den_o,
               acc_sh, ss_t, sd_t, den_t, srcb, dstb, eb, rows, zbuf, gb):
  c = lax.axis_index("c")
  s = lax.axis_index("s")

  pltpu.sync_copy(ss_h, ss_t)
  pltpu.sync_copy(sd_h, sd_t)
  pltpu.sync_copy(g_h, gb)
  gvec = plsc.load_gather(gb, [jnp.zeros((L,), jnp.int32)])

  zero16 = jnp.zeros((L,), jnp.float32)
  iota16 = lax.iota(jnp.int32, L)

  @pl.loop(0, N // L)
  def _zero_den(i):
    plsc.store_scatter(den_t, [i * L + iota16], zero16)

  for i in range(ZR):
    for r in range(D // L):
      plsc.store_scatter(zbuf, [jnp.full((L,), i, jnp.int32), r * L + iota16],
                         zero16)
  rpt = N // NS
  row0 = s * rpt

  @pl.loop(0, rpt // ZR)
  def _zero_acc(i):
    pltpu.sync_copy(zbuf, acc_sh.at[pl.ds(row0 + i * ZR, ZR)])

  plsc.subcore_barrier()

  base0 = c * EPC + s * EW

  @pl.loop(0, NCHUNK)
  def _chunk(k):
    base = base0 + k * C
    pltpu.sync_copy(src_h.at[pl.ds(base, C)], srcb)
    pltpu.sync_copy(dst_h.at[pl.ds(base, C)], dstb)
    pltpu.sync_copy(e_h.at[pl.ds(base, C)], eb)
    pltpu.sync_copy(h_h.at[srcb], rows)

    @pl.loop(0, C // L)
    def _grp(j):
      off = j * L
      si = srcb[pl.ds(off, L)]
      di = dstb[pl.ds(off, L)]
      ev = eb[pl.ds(off, L)]
      alpha = plsc.load_gather(ss_t, [si]) + plsc.load_gather(sd_t, [di]) + ev
      ex = jnp.exp(_lrelu(alpha) - gvec)
      plsc.addupdate_scatter(den_t, [di], ex)
      for i in range(L):
        coef = jnp.broadcast_to(ex[i], (L,))
        ridx = jnp.full((L,), off + i, jnp.int32)
        for r in range(D // L):
          cidx = r * L + iota16
          v = plsc.load_gather(rows, [ridx, cidx])
          plsc.store_scatter(rows, [ridx, cidx], v * coef)

    pltpu.sync_copy(rows, acc_sh.at[dstb], add=True)

  plsc.subcore_barrier()

  pltpu.sync_copy(acc_sh.at[pl.ds(row0, rpt)], acc_o.at[c, pl.ds(row0, rpt)])
  pltpu.sync_copy(den_t, den_o.at[c, s])


_edge_pass = pl.kernel(
    _edge_body,
    out_type=[
        jax.ShapeDtypeStruct((NC, N, D), jnp.float32),
        jax.ShapeDtypeStruct((NC, NS, N), jnp.float32),
    ],
    mesh=plsc.VectorSubcoreMesh(core_axis_name="c", subcore_axis_name="s",
                                num_cores=NC, num_subcores=NS),
    scratch_types=[
        pltpu.VMEM_SHARED((N, D), jnp.float32),
        pltpu.VMEM((N,), jnp.float32),
        pltpu.VMEM((N,), jnp.float32),
        pltpu.VMEM((N,), jnp.float32),
        pltpu.VMEM((C,), jnp.int32),
        pltpu.VMEM((C,), jnp.int32),
        pltpu.VMEM((C,), jnp.float32),
        pltpu.VMEM((C, D), jnp.float32),
        pltpu.VMEM((ZR, D), jnp.float32),
        pltpu.VMEM((L,), jnp.float32),
    ],
)


@jax.jit
def kernel(x, x_ext, edge_index, edge_weight, W1, att_src1, att_dst1, We1,
           att_e1, b1, W2, att_src2, att_dst2, We2, att_e2, b2, W_lin, b_lin):
  src = edge_index[0]
  dst = edge_index[1]
  ewT = edge_weight.T
  w1a = W1[:, :D].T
  w1b = W1[:, D:].T
  w2t = W2.T
  wla = W_lin[:, :H].T
  wlb = W_lin[:, H:].T

  f32 = jnp.float32
  prep = pl.pallas_call(
      _prep_body,
      out_shape=[
          jax.ShapeDtypeStruct((N, D), f32),
          jax.ShapeDtypeStruct((N, 1), f32),
          jax.ShapeDtypeStruct((N, 1), f32),
          jax.ShapeDtypeStruct((E,), f32),
          jax.ShapeDtypeStruct((E,), f32),
          jax.ShapeDtypeStruct((L,), f32),
          jax.ShapeDtypeStruct((1, 1), f32),
          jax.ShapeDtypeStruct((1, 1), f32),
          jax.ShapeDtypeStruct((1, 1), f32),
      ],
      in_specs=[_VMEM_SPEC] * 11,
      out_specs=[_VMEM_SPEC] * 6 + [_SMEM_SPEC] * 3,
  )
  h1, ss1, sd1, e1, e2, g1v, c1, c2, m2 = prep(
      x, x_ext, ewT, w1a, w1b, att_src1, att_dst1, We1, att_e1, We2, att_e2)

  acc1, den1 = _edge_pass(src, dst, e1, ss1.reshape(N), sd1.reshape(N), h1,
                          g1v)

  g1s = g1v[:1].reshape(1, 1)
  mid = pl.pallas_call(
      _mid_body,
      out_shape=[
          jax.ShapeDtypeStruct((N, D), f32),
          jax.ShapeDtypeStruct((N, 1), f32),
          jax.ShapeDtypeStruct((N, 1), f32),
          jax.ShapeDtypeStruct((L,), f32),
          jax.ShapeDtypeStruct((1, 1), f32),
      ],
      in_specs=[_VMEM_SPEC] * 5 + [_SMEM_SPEC] * 3 + [_VMEM_SPEC] * 4,
      out_specs=[_VMEM_SPEC] * 4 + [_SMEM_SPEC],
  )
  h2, ss2, sd2, g2v, g2 = mid(acc1, den1, h1, ss1, sd1, c1, g1s, m2, b1, w2t,
                              att_src2, att_dst2)

  acc2, den2 = _edge_pass(src, dst, e2, ss2.reshape(N), sd2.reshape(N), h2,
                          g2v)

  fin = pl.pallas_call(
      _final_body,
      out_shape=jax.ShapeDtypeStruct((N, 2), f32),
      in_specs=[_VMEM_SPEC] * 5 + [_SMEM_SPEC] * 2 + [_VMEM_SPEC] * 5,
      out_specs=_VMEM_SPEC,
  )
  out = fin(acc2, den2, h2, ss2, sd2, c2, g2, b2, x_ext, wla, wlb, b_lin)
  return out


# speedup vs baseline: 14.5097x; 14.5097x over previous
import jax
import jax.numpy as jnp
from jax import lax
from jax.experimental import pallas as pl
from jax.experimental.pallas import tpu as pltpu
from jax.experimental.pallas import tpu_sc as plsc

N = 10000
E = 320000
D = 128
XE = 3
H = 128
ED = 4

NC = 2
NS = 16
L = 16

EPC = E // NC
EW = E // (NC * NS)
C = 80
NROW = E // C
RPW = EW // C
CB = 5
NBIG = RPW // CB
RPT = 632
NP = NS * RPT
ZR = 8

_SLOPE = 0.2

_VMEM_SPEC = pl.BlockSpec(memory_space=pltpu.MemorySpace.VMEM)
_SMEM_SPEC = pl.BlockSpec(memory_space=pltpu.MemorySpace.SMEM)


def _lrelu(x):
  return jnp.where(x >= 0, x, _SLOPE * x)


def _prep_body(x_ref, xe_ref, ewT_ref, w1a_ref, w1b_ref, as1_ref, ad1_ref,
               we1_ref, ae1_ref, we2_ref, ae2_ref,
               h1_ref, ss1_ref, sd1_ref, e1_ref, e2_ref,
               g1v_ref, c1_ref, c2_ref, m2_ref):
  x = x_ref[...]
  xe = xe_ref[...]
  h1 = jnp.dot(x, w1a_ref[...], preferred_element_type=jnp.float32)
  h1 = h1 + jnp.dot(xe, w1b_ref[...], preferred_element_type=jnp.float32)
  h1_ref[...] = h1
  ss1 = jnp.sum(h1 * as1_ref[...][None, :], axis=1, keepdims=True)
  sd1 = jnp.sum(h1 * ad1_ref[...][None, :], axis=1, keepdims=True)
  ss1_ref[...] = ss1
  sd1_ref[...] = sd1
  wvec1 = jnp.sum(we1_ref[...] * ae1_ref[...][:, None], axis=0)
  wvec2 = jnp.sum(we2_ref[...] * ae2_ref[...][:, None], axis=0)
  ewT = ewT_ref[...]
  e1 = jnp.sum(ewT * wvec1[:, None], axis=0)
  e2 = jnp.sum(ewT * wvec2[:, None], axis=0)
  e1_ref[...] = e1
  e2_ref[...] = e2
  c1 = jnp.mean(e1)
  c2 = jnp.mean(e2)
  m1 = jnp.maximum(jnp.max(e1), c1)
  m2 = jnp.maximum(jnp.max(e2), c2)
  g1 = _lrelu(jnp.max(ss1) + jnp.max(sd1) + m1)
  g1v_ref[...] = jnp.full((L,), g1, jnp.float32)
  c1_ref[0, 0] = c1
  c2_ref[0, 0] = c2
  m2_ref[0, 0] = m2


def _combine(acc_ref, den_ref, h_ref, ss_ref, sd_ref, cc, gg, b_ref):
  exl = jnp.exp(_lrelu(ss_ref[...] + sd_ref[...] + cc) - gg)
  den = jnp.sum(den_ref[...], axis=(0, 1))[:, None] + exl + 1e-16
  num = acc_ref[0, :N] + acc_ref[1, :N] + exl * h_ref[...]
  return jax.nn.relu(num / den + b_ref[...][None, :])


def _mid_body(acc_ref, den_ref, h1_ref, ss1_ref, sd1_ref, c1_ref, g1_ref,
              m2_ref, b1_ref, w2_ref, as2_ref, ad2_ref,
              h2_ref, ss2_ref, sd2_ref, g2v_ref, g2_ref):
  x2 = _combine(acc_ref, den_ref, h1_ref, ss1_ref, sd1_ref,
                c1_ref[0, 0], g1_ref[0, 0], b1_ref)
  h2 = jnp.dot(x2, w2_ref[...], preferred_element_type=jnp.float32)
  h2_ref[...] = h2
  ss2 = jnp.sum(h2 * as2_ref[...][None, :], axis=1, keepdims=True)
  sd2 = jnp.sum(h2 * ad2_ref[...][None, :], axis=1, keepdims=True)
  ss2_ref[...] = ss2
  sd2_ref[...] = sd2
  g2 = _lrelu(jnp.max(ss2) + jnp.max(sd2) + m2_ref[0, 0])
  g2v_ref[...] = jnp.full((L,), g2, jnp.float32)
  g2_ref[0, 0] = g2


def _final_body(acc_ref, den_ref, h2_ref, ss2_ref, sd2_ref, c2_ref, g2_ref,
                b2_ref, xe_ref, wla_ref, wlb_ref, bl_ref, out_ref):
  x3 = _combine(acc_ref, den_ref, h2_ref, ss2_ref, sd2_ref,
                c2_ref[0, 0], g2_ref[0, 0], b2_ref)
  z = jnp.dot(x3, wla_ref[...], preferred_element_type=jnp.float32)
  z = z + jnp.dot(xe_ref[...], wlb_ref[...], preferred_element_type=jnp.float32)
  out_ref[...] = jax.nn.relu(z + bl_ref[...][None, :])


def _edge_body(src_h, dst_h, e_h, ss_h, sd_h, h_h, g_h,
               acc_o, den_o,
               acc_sh, ss_t, sd_t, den_t, srcb, dstb, eb, rows, zbuf, gb):
  c = lax.axis_index("c")
  s = lax.axis_index("s")

  pltpu.sync_copy(ss_h, ss_t)
  pltpu.sync_copy(sd_h, sd_t)
  pltpu.sync_copy(g_h, gb)
  gvec = plsc.load_gather(gb, [jnp.zeros((L,), jnp.int32)])

  zero16 = jnp.zeros((L,), jnp.float32)
  iota16 = lax.iota(jnp.int32, L)
  zeroi16 = jnp.zeros((L,), jnp.int32)

  @pl.loop(0, N // L)
  def _zero_den(i):
    plsc.store_scatter(den_t, [zeroi16, i * L + iota16], zero16)

  for i in range(ZR):
    for r in range(D // L):
      plsc.store_scatter(zbuf, [jnp.full((L,), i, jnp.int32), r * L + iota16],
                         zero16)
  row0 = s * RPT

  @pl.loop(0, RPT // ZR)
  def _zero_acc(i):
    pltpu.sync_copy(zbuf, acc_sh.at[pl.ds(row0 + i * ZR, ZR)])

  plsc.subcore_barrier()

  w = c * NS + s

  @pl.loop(0, NBIG)
  def _big(g):
    pltpu.sync_copy(src_h.at[w, g], srcb)
    pltpu.sync_copy(dst_h.at[w, g], dstb)
    pltpu.sync_copy(e_h.at[w, g], eb)

    @pl.loop(0, CB)
    def _sub(j):
      pltpu.sync_copy(h_h.at[srcb.at[j]], rows)

      @pl.loop(0, C // L)
      def _grp(q):
        off = q * L
        si = srcb[j, pl.ds(off, L)]
        di = dstb[j, pl.ds(off, L)]
        ev = eb[j, pl.ds(off, L)]
        alpha = (plsc.load_gather(ss_t, [si]) + plsc.load_gather(sd_t, [di])
                 + ev)
        ex = jnp.exp(_lrelu(alpha) - gvec)
        plsc.addupdate_scatter(den_t, [zeroi16, di], ex)
        for i in range(L):
          coef = jnp.broadcast_to(ex[i], (L,))
          ridx = jnp.full((L,), off + i, jnp.int32)
          for r in range(D // L):
            cidx = r * L + iota16
            v = plsc.load_gather(rows, [ridx, cidx])
            plsc.store_scatter(rows, [ridx, cidx], v * coef)

      pltpu.sync_copy(rows, acc_sh.at[dstb.at[j]], add=True)

  plsc.subcore_barrier()

  pltpu.sync_copy(acc_sh.at[pl.ds(row0, RPT)], acc_o.at[c, pl.ds(row0, RPT)])
  pltpu.sync_copy(den_t, den_o.at[w])


_edge_pass = pl.kernel(
    _edge_body,
    out_type=[
        jax.ShapeDtypeStruct((NC, NP, D), jnp.float32),
        jax.ShapeDtypeStruct((NC * NS, 1, N), jnp.float32),
    ],
    mesh=plsc.VectorSubcoreMesh(core_axis_name="c", subcore_axis_name="s",
                                num_cores=NC, num_subcores=NS),
    compiler_params=pltpu.CompilerParams(needs_layout_passes=False),
    scratch_types=[
        pltpu.VMEM_SHARED((NP, D), jnp.float32),
        pltpu.VMEM((N,), jnp.float32),
        pltpu.VMEM((N,), jnp.float32),
        pltpu.VMEM((1, N), jnp.float32),
        pltpu.VMEM((CB, C), jnp.int32),
        pltpu.VMEM((CB, C), jnp.int32),
        pltpu.VMEM((CB, C), jnp.float32),
        pltpu.VMEM((C, D), jnp.float32),
        pltpu.VMEM((ZR, D), jnp.float32),
        pltpu.VMEM((L,), jnp.float32),
    ],
)


@jax.jit
def kernel(x, x_ext, edge_index, edge_weight, W1, att_src1, att_dst1, We1,
           att_e1, b1, W2, att_src2, att_dst2, We2, att_e2, b2, W_lin, b_lin):
  src = edge_index[0]
  dst = edge_index[1]
  ewT = edge_weight.T
  w1a = W1[:, :D].T
  w1b = W1[:, D:].T
  w2t = W2.T
  wla = W_lin[:, :H].T
  wlb = W_lin[:, H:].T

  f32 = jnp.float32
  prep = pl.pallas_call(
      _prep_body,
      out_shape=[
          jax.ShapeDtypeStruct((N, D), f32),
          jax.ShapeDtypeStruct((N, 1), f32),
          jax.ShapeDtypeStruct((N, 1), f32),
          jax.ShapeDtypeStruct((E,), f32),
          jax.ShapeDtypeStruct((E,), f32),
          jax.ShapeDtypeStruct((L,), f32),
          jax.ShapeDtypeStruct((1, 1), f32),
          jax.ShapeDtypeStruct((1, 1), f32),
          jax.ShapeDtypeStruct((1, 1), f32),
      ],
      in_specs=[_VMEM_SPEC] * 11,
      out_specs=[_VMEM_SPEC] * 6 + [_SMEM_SPEC] * 3,
  )
  h1, ss1, sd1, e1, e2, g1v, c1, c2, m2 = prep(
      x, x_ext, ewT, w1a, w1b, att_src1, att_dst1, We1, att_e1, We2, att_e2)

  eshape = (NC * NS, NBIG, CB, C)
  src2 = src.reshape(eshape)
  dst2 = dst.reshape(eshape)
  acc1, den1 = _edge_pass(src2, dst2, e1.reshape(eshape), ss1.reshape(N),
                          sd1.reshape(N), h1, g1v)

  g1s = g1v[:1].reshape(1, 1)
  mid = pl.pallas_call(
      _mid_body,
      out_shape=[
          jax.ShapeDtypeStruct((N, D), f32),
          jax.ShapeDtypeStruct((N, 1), f32),
          jax.ShapeDtypeStruct((N, 1), f32),
          jax.ShapeDtypeStruct((L,), f32),
          jax.ShapeDtypeStruct((1, 1), f32),
      ],
      in_specs=[_VMEM_SPEC] * 5 + [_SMEM_SPEC] * 3 + [_VMEM_SPEC] * 4,
      out_specs=[_VMEM_SPEC] * 4 + [_SMEM_SPEC],
  )
  h2, ss2, sd2, g2v, g2 = mid(acc1, den1, h1, ss1, sd1, c1, g1s, m2, b1, w2t,
                              att_src2, att_dst2)

  acc2, den2 = _edge_pass(src2, dst2, e2.reshape(eshape), ss2.reshape(N),
                          sd2.reshape(N), h2, g2v)

  fin = pl.pallas_call(
      _final_body,
      out_shape=jax.ShapeDtypeStruct((N, 2), f32),
      in_specs=[_VMEM_SPEC] * 5 + [_SMEM_SPEC] * 2 + [_VMEM_SPEC] * 5,
      out_specs=_VMEM_SPEC,
  )
  out = fin(acc2, den2, h2, ss2, sd2, c2, g2, b2, x_ext, wla, wlb, b_lin)
  return out


# speedup vs baseline: 26.3462x; 1.8158x over previous
import jax
import jax.numpy as jnp
from jax import lax
from jax.experimental import pallas as pl
from jax.experimental.pallas import tpu as pltpu
from jax.experimental.pallas import tpu_sc as plsc

N = 10000
E = 320000
D = 128
XE = 3
H = 128
ED = 4

NC = 2
NS = 16
L = 16

EPC = E // NC
EW = E // (NC * NS)
C = 80
NROW = E // C
RPW = EW // C
CB = 5
NBIG = RPW // CB
RPT = 632
NP = NS * RPT
ZR = 8

_SLOPE = 0.2

_VMEM_SPEC = pl.BlockSpec(memory_space=pltpu.MemorySpace.VMEM)
_SMEM_SPEC = pl.BlockSpec(memory_space=pltpu.MemorySpace.SMEM)


def _lrelu(x):
  return jnp.where(x >= 0, x, _SLOPE * x)


def _prep_body(x_ref, xe_ref, ewT_ref, w1a_ref, w1b_ref, as1_ref, ad1_ref,
               we1_ref, ae1_ref, we2_ref, ae2_ref,
               h1_ref, ss1_ref, sd1_ref, e1_ref, e2_ref,
               g1v_ref, c1_ref, c2_ref, m2_ref):
  x = x_ref[...]
  xe = xe_ref[...]
  h1 = jnp.dot(x, w1a_ref[...], preferred_element_type=jnp.float32)
  h1 = h1 + jnp.dot(xe, w1b_ref[...], preferred_element_type=jnp.float32)
  h1_ref[...] = h1
  ss1 = jnp.sum(h1 * as1_ref[...][None, :], axis=1, keepdims=True)
  sd1 = jnp.sum(h1 * ad1_ref[...][None, :], axis=1, keepdims=True)
  ss1_ref[...] = ss1
  sd1_ref[...] = sd1
  wvec1 = jnp.sum(we1_ref[...] * ae1_ref[...][:, None], axis=0)
  wvec2 = jnp.sum(we2_ref[...] * ae2_ref[...][:, None], axis=0)
  ewT = ewT_ref[...]
  e1 = jnp.sum(ewT * wvec1[:, None], axis=0)
  e2 = jnp.sum(ewT * wvec2[:, None], axis=0)
  e1_ref[...] = e1
  e2_ref[...] = e2
  c1 = jnp.mean(e1)
  c2 = jnp.mean(e2)
  m1 = jnp.maximum(jnp.max(e1), c1)
  m2 = jnp.maximum(jnp.max(e2), c2)
  g1 = _lrelu(jnp.max(ss1) + jnp.max(sd1) + m1)
  g1v_ref[...] = jnp.full((L,), g1, jnp.float32)
  c1_ref[0, 0] = c1
  c2_ref[0, 0] = c2
  m2_ref[0, 0] = m2


def _combine(acc_ref, den_ref, h_ref, ss_ref, sd_ref, cc, gg, b_ref):
  exl = jnp.exp(_lrelu(ss_ref[...] + sd_ref[...] + cc) - gg)
  den = jnp.sum(den_ref[...], axis=(0, 1))[:, None] + exl + 1e-16
  num = acc_ref[0, :N] + acc_ref[1, :N] + exl * h_ref[...]
  return jax.nn.relu(num / den + b_ref[...][None, :])


def _mid_body(acc_ref, den_ref, h1_ref, ss1_ref, sd1_ref, c1_ref, g1_ref,
              m2_ref, b1_ref, w2_ref, as2_ref, ad2_ref,
              h2_ref, ss2_ref, sd2_ref, g2v_ref, g2_ref):
  x2 = _combine(acc_ref, den_ref, h1_ref, ss1_ref, sd1_ref,
                c1_ref[0, 0], g1_ref[0, 0], b1_ref)
  h2 = jnp.dot(x2, w2_ref[...], preferred_element_type=jnp.float32)
  h2_ref[...] = h2
  ss2 = jnp.sum(h2 * as2_ref[...][None, :], axis=1, keepdims=True)
  sd2 = jnp.sum(h2 * ad2_ref[...][None, :], axis=1, keepdims=True)
  ss2_ref[...] = ss2
  sd2_ref[...] = sd2
  g2 = _lrelu(jnp.max(ss2) + jnp.max(sd2) + m2_ref[0, 0])
  g2v_ref[...] = jnp.full((L,), g2, jnp.float32)
  g2_ref[0, 0] = g2


def _final_body(acc_ref, den_ref, h2_ref, ss2_ref, sd2_ref, c2_ref, g2_ref,
                b2_ref, xe_ref, wla_ref, wlb_ref, bl_ref, out_ref):
  x3 = _combine(acc_ref, den_ref, h2_ref, ss2_ref, sd2_ref,
                c2_ref[0, 0], g2_ref[0, 0], b2_ref)
  z = jnp.dot(x3, wla_ref[...], preferred_element_type=jnp.float32)
  z = z + jnp.dot(xe_ref[...], wlb_ref[...], preferred_element_type=jnp.float32)
  out_ref[...] = jax.nn.relu(z + bl_ref[...][None, :])


def _edge_body(src_h, dst_h, e_h, ss_h, sd_h, h_h, g_h,
               acc_o, den_o,
               acc_sh, ss_t, sd_t, den_t, srcb, dstb, eb, rows, zbuf, gb):
  c = lax.axis_index("c")
  s = lax.axis_index("s")

  pltpu.sync_copy(ss_h, ss_t)
  pltpu.sync_copy(sd_h, sd_t)
  pltpu.sync_copy(g_h, gb)
  gvec = plsc.load_gather(gb, [jnp.zeros((L,), jnp.int32)])

  zero16 = jnp.zeros((L,), jnp.float32)
  iota16 = lax.iota(jnp.int32, L)
  zeroi16 = jnp.zeros((L,), jnp.int32)

  @pl.loop(0, N // L)
  def _zero_den(i):
    plsc.store_scatter(den_t, [zeroi16, i * L + iota16], zero16)

  for i in range(ZR):
    for r in range(D // L):
      plsc.store_scatter(zbuf, [jnp.full((L,), i, jnp.int32), r * L + iota16],
                         zero16)
  row0 = s * RPT

  @pl.loop(0, RPT // ZR)
  def _zero_acc(i):
    pltpu.sync_copy(zbuf, acc_sh.at[pl.ds(row0 + i * ZR, ZR)])

  plsc.subcore_barrier()

  w = c * NS + s

  @pl.loop(0, NBIG)
  def _big(g):
    pltpu.sync_copy(src_h.at[w, g], srcb)
    pltpu.sync_copy(dst_h.at[w, g], dstb)
    pltpu.sync_copy(e_h.at[w, g], eb)

    @pl.loop(0, CB)
    def _sub(j):
      pltpu.sync_copy(h_h.at[srcb.at[j]], rows)

      for q in range(C // L):
        off = q * L
        si = srcb[j, pl.ds(off, L)]
        di = dstb[j, pl.ds(off, L)]
        ev = eb[j, pl.ds(off, L)]
        alpha = (plsc.load_gather(ss_t, [si]) + plsc.load_gather(sd_t, [di])
                 + ev)
        ex = jnp.exp(_lrelu(alpha) - gvec)
        plsc.addupdate_scatter(den_t, [zeroi16, di], ex)
        for i in range(L):
          coef = jnp.broadcast_to(ex[i], (L,))
          for r in range(D // L):
            rows[off + i, pl.ds(r * L, L)] = (
                rows[off + i, pl.ds(r * L, L)] * coef)

      pltpu.sync_copy(rows, acc_sh.at[dstb.at[j]], add=True)

  plsc.subcore_barrier()

  pltpu.sync_copy(acc_sh.at[pl.ds(row0, RPT)], acc_o.at[c, pl.ds(row0, RPT)])
  pltpu.sync_copy(den_t, den_o.at[w])


_edge_pass = pl.kernel(
    _edge_body,
    out_type=[
        jax.ShapeDtypeStruct((NC, NP, D), jnp.float32),
        jax.ShapeDtypeStruct((NC * NS, 1, N), jnp.float32),
    ],
    mesh=plsc.VectorSubcoreMesh(core_axis_name="c", subcore_axis_name="s",
                                num_cores=NC, num_subcores=NS),
    compiler_params=pltpu.CompilerParams(needs_layout_passes=False),
    scratch_types=[
        pltpu.VMEM_SHARED((NP, D), jnp.float32),
        pltpu.VMEM((N,), jnp.float32),
        pltpu.VMEM((N,), jnp.float32),
        pltpu.VMEM((1, N), jnp.float32),
        pltpu.VMEM((CB, C), jnp.int32),
        pltpu.VMEM((CB, C), jnp.int32),
        pltpu.VMEM((CB, C), jnp.float32),
        pltpu.VMEM((C, D), jnp.float32),
        pltpu.VMEM((ZR, D), jnp.float32),
        pltpu.VMEM((L,), jnp.float32),
    ],
)


@jax.jit
def kernel(x, x_ext, edge_index, edge_weight, W1, att_src1, att_dst1, We1,
           att_e1, b1, W2, att_src2, att_dst2, We2, att_e2, b2, W_lin, b_lin):
  src = edge_index[0]
  dst = edge_index[1]
  ewT = edge_weight.T
  w1a = W1[:, :D].T
  w1b = W1[:, D:].T
  w2t = W2.T
  wla = W_lin[:, :H].T
  wlb = W_lin[:, H:].T

  f32 = jnp.float32
  prep = pl.pallas_call(
      _prep_body,
      out_shape=[
          jax.ShapeDtypeStruct((N, D), f32),
          jax.ShapeDtypeStruct((N, 1), f32),
          jax.ShapeDtypeStruct((N, 1), f32),
          jax.ShapeDtypeStruct((E,), f32),
          jax.ShapeDtypeStruct((E,), f32),
          jax.ShapeDtypeStruct((L,), f32),
          jax.ShapeDtypeStruct((1, 1), f32),
          jax.ShapeDtypeStruct((1, 1), f32),
          jax.ShapeDtypeStruct((1, 1), f32),
      ],
      in_specs=[_VMEM_SPEC] * 11,
      out_specs=[_VMEM_SPEC] * 6 + [_SMEM_SPEC] * 3,
  )
  h1, ss1, sd1, e1, e2, g1v, c1, c2, m2 = prep(
      x, x_ext, ewT, w1a, w1b, att_src1, att_dst1, We1, att_e1, We2, att_e2)

  eshape = (NC * NS, NBIG, CB, C)
  src2 = src.reshape(eshape)
  dst2 = dst.reshape(eshape)
  acc1, den1 = _edge_pass(src2, dst2, e1.reshape(eshape), ss1.reshape(N),
                          sd1.reshape(N), h1, g1v)

  g1s = g1v[:1].reshape(1, 1)
  mid = pl.pallas_call(
      _mid_body,
      out_shape=[
          jax.ShapeDtypeStruct((N, D), f32),
          jax.ShapeDtypeStruct((N, 1), f32),
          jax.ShapeDtypeStruct((N, 1), f32),
          jax.ShapeDtypeStruct((L,), f32),
          jax.ShapeDtypeStruct((1, 1), f32),
      ],
      in_specs=[_VMEM_SPEC] * 5 + [_SMEM_SPEC] * 3 + [_VMEM_SPEC] * 4,
      out_specs=[_VMEM_SPEC] * 4 + [_SMEM_SPEC],
  )
  h2, ss2, sd2, g2v, g2 = mid(acc1, den1, h1, ss1, sd1, c1, g1s, m2, b1, w2t,
                              att_src2, att_dst2)

  acc2, den2 = _edge_pass(src2, dst2, e2.reshape(eshape), ss2.reshape(N),
                          sd2.reshape(N), h2, g2v)

  fin = pl.pallas_call(
      _final_body,
      out_shape=jax.ShapeDtypeStruct((N, 2), f32),
      in_specs=[_VMEM_SPEC] * 5 + [_SMEM_SPEC] * 2 + [_VMEM_SPEC] * 5,
      out_specs=_VMEM_SPEC,
  )
  out = fin(acc2, den2, h2, ss2, sd2, c2, g2, b2, x_ext, wla, wlb, b_lin)
  return out


# speedup vs baseline: 27.3262x; 1.0372x over previous
import jax
import jax.numpy as jnp
from jax import lax
from jax.experimental import pallas as pl
from jax.experimental.pallas import tpu as pltpu
from jax.experimental.pallas import tpu_sc as plsc

N = 10000
E = 320000
D = 128
XE = 3
H = 128
ED = 4

NC = 2
NS = 16
L = 16

EPC = E // NC
EW = E // (NC * NS)
C = 80
NROW = E // C
RPW = EW // C
CB = 5
NBIG = RPW // CB
RPT = 632
NP = NS * RPT
ZR = 8
DEN_T = 640

_SLOPE = 0.2

_VMEM_SPEC = pl.BlockSpec(memory_space=pltpu.MemorySpace.VMEM)
_SMEM_SPEC = pl.BlockSpec(memory_space=pltpu.MemorySpace.SMEM)


def _lrelu(x):
  return jnp.where(x >= 0, x, _SLOPE * x)


def _prep_body(x_ref, xe_ref, ewT_ref, w1a_ref, w1b_ref, as1_ref, ad1_ref,
               we1_ref, ae1_ref, we2_ref, ae2_ref,
               h1_ref, ss1_ref, sd1_ref, e1_ref, e2_ref,
               g1v_ref, c1_ref, c2_ref, m2_ref):
  x = x_ref[...]
  xe = xe_ref[...]
  h1 = jnp.dot(x, w1a_ref[...], preferred_element_type=jnp.float32)
  h1 = h1 + jnp.dot(xe, w1b_ref[...], preferred_element_type=jnp.float32)
  h1_ref[...] = h1
  ss1 = jnp.sum(h1 * as1_ref[...][None, :], axis=1, keepdims=True)
  sd1 = jnp.sum(h1 * ad1_ref[...][None, :], axis=1, keepdims=True)
  ss1_ref[...] = ss1
  sd1_ref[...] = sd1
  wvec1 = jnp.sum(we1_ref[...] * ae1_ref[...][:, None], axis=0)
  wvec2 = jnp.sum(we2_ref[...] * ae2_ref[...][:, None], axis=0)
  ewT = ewT_ref[...]
  e1 = jnp.sum(ewT * wvec1[:, None], axis=0)
  e2 = jnp.sum(ewT * wvec2[:, None], axis=0)
  e1_ref[...] = e1
  e2_ref[...] = e2
  c1 = jnp.mean(e1)
  c2 = jnp.mean(e2)
  m1 = jnp.maximum(jnp.max(e1), c1)
  m2 = jnp.maximum(jnp.max(e2), c2)
  g1 = _lrelu(jnp.max(ss1) + jnp.max(sd1) + m1)
  g1v_ref[...] = jnp.full((L,), g1, jnp.float32)
  c1_ref[0, 0] = c1
  c2_ref[0, 0] = c2
  m2_ref[0, 0] = m2


def _combine(acc_ref, den_ref, h_ref, ss_ref, sd_ref, cc, gg, b_ref):
  exl = jnp.exp(_lrelu(ss_ref[...] + sd_ref[...] + cc) - gg)
  den = (den_ref[0, :N] + den_ref[1, :N])[:, None] + exl + 1e-16
  num = acc_ref[0, :N] + acc_ref[1, :N] + exl * h_ref[...]
  return jax.nn.relu(num / den + b_ref[...][None, :])


def _mid_body(acc_ref, den_ref, h1_ref, ss1_ref, sd1_ref, c1_ref, g1_ref,
              m2_ref, b1_ref, w2_ref, as2_ref, ad2_ref,
              h2_ref, ss2_ref, sd2_ref, g2v_ref, g2_ref):
  x2 = _combine(acc_ref, den_ref, h1_ref, ss1_ref, sd1_ref,
                c1_ref[0, 0], g1_ref[0, 0], b1_ref)
  h2 = jnp.dot(x2, w2_ref[...], preferred_element_type=jnp.float32)
  h2_ref[...] = h2
  ss2 = jnp.sum(h2 * as2_ref[...][None, :], axis=1, keepdims=True)
  sd2 = jnp.sum(h2 * ad2_ref[...][None, :], axis=1, keepdims=True)
  ss2_ref[...] = ss2
  sd2_ref[...] = sd2
  g2 = _lrelu(jnp.max(ss2) + jnp.max(sd2) + m2_ref[0, 0])
  g2v_ref[...] = jnp.full((L,), g2, jnp.float32)
  g2_ref[0, 0] = g2


def _final_body(acc_ref, den_ref, h2_ref, ss2_ref, sd2_ref, c2_ref, g2_ref,
                b2_ref, xe_ref, wla_ref, wlb_ref, bl_ref, out_ref):
  x3 = _combine(acc_ref, den_ref, h2_ref, ss2_ref, sd2_ref,
                c2_ref[0, 0], g2_ref[0, 0], b2_ref)
  z = jnp.dot(x3, wla_ref[...], preferred_element_type=jnp.float32)
  z = z + jnp.dot(xe_ref[...], wlb_ref[...], preferred_element_type=jnp.float32)
  out_ref[...] = jax.nn.relu(z + bl_ref[...][None, :])


def _edge_body(src_h, dst_h, e_h, ss_h, sd_h, h_h, g_h,
               acc_o, den_o,
               acc_sh, den_sh, ss_t, sd_t, srcb, dstb, eb, rows, zbuf, gb,
               exb, z1, gsem0, gsem1, ssem0, ssem1):
  gsem = (gsem0, gsem1)
  ssem = (ssem0, ssem1)
  c = lax.axis_index("c")
  s = lax.axis_index("s")

  pltpu.sync_copy(ss_h, ss_t)
  pltpu.sync_copy(sd_h, sd_t)
  pltpu.sync_copy(g_h, gb)
  gvec = plsc.load_gather(gb, [jnp.zeros((L,), jnp.int32)])

  zero16 = jnp.zeros((L,), jnp.float32)
  iota16 = lax.iota(jnp.int32, L)

  for i in range(ZR):
    for r in range(D // L):
      plsc.store_scatter(zbuf, [jnp.full((L,), i, jnp.int32), r * L + iota16],
                         zero16)
  for k in range(DEN_T // L):
    plsc.store_scatter(z1, [k * L + iota16], zero16)
  row0 = s * RPT

  @pl.loop(0, RPT // ZR)
  def _zero_acc(i):
    pltpu.sync_copy(zbuf, acc_sh.at[pl.ds(row0 + i * ZR, ZR)])

  pltpu.sync_copy(z1, den_sh.at[pl.ds(s * DEN_T, DEN_T)])

  plsc.subcore_barrier()

  w = c * NS + s

  @pl.loop(0, NBIG)
  def _big(g):
    pltpu.sync_copy(src_h.at[w, g], srcb)
    pltpu.sync_copy(dst_h.at[w, g], dstb)
    pltpu.sync_copy(e_h.at[w, g], eb)

    pltpu.make_async_copy(h_h.at[srcb.at[0]], rows.at[0], gsem[0]).start()
    for j in range(CB):
      b = j & 1
      if j + 1 < CB:
        if j >= 1:
          pltpu.make_async_copy(rows.at[1 - b], acc_sh.at[dstb.at[j - 1]],
                                ssem[1 - b]).wait()
        pltpu.make_async_copy(h_h.at[srcb.at[j + 1]], rows.at[1 - b],
                              gsem[1 - b]).start()
      pltpu.make_async_copy(h_h.at[srcb.at[j]], rows.at[b], gsem[b]).wait()

      for q in range(C // L):
        off = q * L
        si = srcb[j, pl.ds(off, L)]
        di = dstb[j, pl.ds(off, L)]
        ev = eb[j, pl.ds(off, L)]
        alpha = (plsc.load_gather(ss_t, [si]) + plsc.load_gather(sd_t, [di])
                 + ev)
        ex = jnp.exp(_lrelu(alpha) - gvec)
        exb[pl.ds(off, L)] = ex
        for i in range(L):
          coef = jnp.broadcast_to(ex[i], (L,))
          for r in range(D // L):
            rows[b, off + i, pl.ds(r * L, L)] = (
                rows[b, off + i, pl.ds(r * L, L)] * coef)

      pltpu.make_async_copy(rows.at[b], acc_sh.at[dstb.at[j]],
                            ssem[b]).start(add=True)
      pltpu.sync_copy(exb, den_sh.at[dstb.at[j]], add=True)

    pltpu.make_async_copy(rows.at[(CB - 2) & 1], acc_sh.at[dstb.at[CB - 2]],
                          ssem[(CB - 2) & 1]).wait()
    pltpu.make_async_copy(rows.at[(CB - 1) & 1], acc_sh.at[dstb.at[CB - 1]],
                          ssem[(CB - 1) & 1]).wait()

  plsc.subcore_barrier()

  pltpu.sync_copy(acc_sh.at[pl.ds(row0, RPT)], acc_o.at[c, pl.ds(row0, RPT)])
  pltpu.sync_copy(den_sh.at[pl.ds(s * DEN_T, DEN_T)],
                  den_o.at[c, pl.ds(s * DEN_T, DEN_T)])


_edge_pass = pl.kernel(
    _edge_body,
    out_type=[
        jax.ShapeDtypeStruct((NC, NP, D), jnp.float32),
        jax.ShapeDtypeStruct((NC, NS * DEN_T), jnp.float32),
    ],
    mesh=plsc.VectorSubcoreMesh(core_axis_name="c", subcore_axis_name="s",
                                num_cores=NC, num_subcores=NS),
    compiler_params=pltpu.CompilerParams(needs_layout_passes=False),
    scratch_types=[
        pltpu.VMEM_SHARED((NP, D), jnp.float32),
        pltpu.VMEM_SHARED((NS * DEN_T,), jnp.float32),
        pltpu.VMEM((N,), jnp.float32),
        pltpu.VMEM((N,), jnp.float32),
        pltpu.VMEM((CB, C), jnp.int32),
        pltpu.VMEM((CB, C), jnp.int32),
        pltpu.VMEM((CB, C), jnp.float32),
        pltpu.VMEM((2, C, D), jnp.float32),
        pltpu.VMEM((ZR, D), jnp.float32),
        pltpu.VMEM((L,), jnp.float32),
        pltpu.VMEM((C,), jnp.float32),
        pltpu.VMEM((DEN_T,), jnp.float32),
        pltpu.SemaphoreType.DMA,
        pltpu.SemaphoreType.DMA,
        pltpu.SemaphoreType.DMA,
        pltpu.SemaphoreType.DMA,
    ],
)


@jax.jit
def kernel(x, x_ext, edge_index, edge_weight, W1, att_src1, att_dst1, We1,
           att_e1, b1, W2, att_src2, att_dst2, We2, att_e2, b2, W_lin, b_lin):
  src = edge_index[0]
  dst = edge_index[1]
  ewT = edge_weight.T
  w1a = W1[:, :D].T
  w1b = W1[:, D:].T
  w2t = W2.T
  wla = W_lin[:, :H].T
  wlb = W_lin[:, H:].T

  f32 = jnp.float32
  prep = pl.pallas_call(
      _prep_body,
      out_shape=[
          jax.ShapeDtypeStruct((N, D), f32),
          jax.ShapeDtypeStruct((N, 1), f32),
          jax.ShapeDtypeStruct((N, 1), f32),
          jax.ShapeDtypeStruct((E,), f32),
          jax.ShapeDtypeStruct((E,), f32),
          jax.ShapeDtypeStruct((L,), f32),
          jax.ShapeDtypeStruct((1, 1), f32),
          jax.ShapeDtypeStruct((1, 1), f32),
          jax.ShapeDtypeStruct((1, 1), f32),
      ],
      in_specs=[_VMEM_SPEC] * 11,
      out_specs=[_VMEM_SPEC] * 6 + [_SMEM_SPEC] * 3,
  )
  h1, ss1, sd1, e1, e2, g1v, c1, c2, m2 = prep(
      x, x_ext, ewT, w1a, w1b, att_src1, att_dst1, We1, att_e1, We2, att_e2)

  eshape = (NC * NS, NBIG, CB, C)
  src2 = src.reshape(eshape)
  dst2 = dst.reshape(eshape)
  acc1, den1 = _edge_pass(src2, dst2, e1.reshape(eshape), ss1.reshape(N),
                          sd1.reshape(N), h1, g1v)

  g1s = g1v[:1].reshape(1, 1)
  mid = pl.pallas_call(
      _mid_body,
      out_shape=[
          jax.ShapeDtypeStruct((N, D), f32),
          jax.ShapeDtypeStruct((N, 1), f32),
          jax.ShapeDtypeStruct((N, 1), f32),
          jax.ShapeDtypeStruct((L,), f32),
          jax.ShapeDtypeStruct((1, 1), f32),
      ],
      in_specs=[_VMEM_SPEC] * 5 + [_SMEM_SPEC] * 3 + [_VMEM_SPEC] * 4,
      out_specs=[_VMEM_SPEC] * 4 + [_SMEM_SPEC],
  )
  h2, ss2, sd2, g2v, g2 = mid(acc1, den1, h1, ss1, sd1, c1, g1s, m2, b1, w2t,
                              att_src2, att_dst2)

  acc2, den2 = _edge_pass(src2, dst2, e2.reshape(eshape), ss2.reshape(N),
                          sd2.reshape(N), h2, g2v)

  fin = pl.pallas_call(
      _final_body,
      out_shape=jax.ShapeDtypeStruct((N, 2), f32),
      in_specs=[_VMEM_SPEC] * 5 + [_SMEM_SPEC] * 2 + [_VMEM_SPEC] * 5,
      out_specs=_VMEM_SPEC,
  )
  out = fin(acc2, den2, h2, ss2, sd2, c2, g2, b2, x_ext, wla, wlb, b_lin)
  return out


# speedup vs baseline: 28.9974x; 1.0612x over previous
import jax
import jax.numpy as jnp
from jax import lax
from jax.experimental import pallas as pl
from jax.experimental.pallas import tpu as pltpu
from jax.experimental.pallas import tpu_sc as plsc

N = 10000
E = 320000
D = 128
XE = 3
H = 128
ED = 4

NC = 2
NS = 16
L = 16

EPC = E // NC
EW = E // (NC * NS)
C = 80
NROW = E // C
RPW = EW // C
CB = 5
NBIG = RPW // CB
RPT = 632
NP = NS * RPT
ZR = 8
DEN_T = 640

_SLOPE = 0.2

_VMEM_SPEC = pl.BlockSpec(memory_space=pltpu.MemorySpace.VMEM)
_SMEM_SPEC = pl.BlockSpec(memory_space=pltpu.MemorySpace.SMEM)


def _lrelu(x):
  return jnp.where(x >= 0, x, _SLOPE * x)


def _prep_body(x_ref, xe_ref, ewT_ref, w1a_ref, w1b_ref, as1_ref, ad1_ref,
               we1_ref, ae1_ref, we2_ref, ae2_ref,
               h1_ref, ss1_ref, sd1_ref, e1_ref, e2_ref,
               g1v_ref, c1_ref, c2_ref, m2_ref):
  x = x_ref[...]
  xe = xe_ref[...]
  h1 = jnp.dot(x, w1a_ref[...], preferred_element_type=jnp.float32)
  h1 = h1 + jnp.dot(xe, w1b_ref[...], preferred_element_type=jnp.float32)
  h1_ref[...] = h1
  ss1 = jnp.sum(h1 * as1_ref[...][None, :], axis=1, keepdims=True)
  sd1 = jnp.sum(h1 * ad1_ref[...][None, :], axis=1, keepdims=True)
  ss1_ref[...] = ss1
  sd1_ref[...] = sd1
  wvec1 = jnp.sum(we1_ref[...] * ae1_ref[...][:, None], axis=0)
  wvec2 = jnp.sum(we2_ref[...] * ae2_ref[...][:, None], axis=0)
  ewT = ewT_ref[...]
  e1 = jnp.sum(ewT * wvec1[:, None], axis=0)
  e2 = jnp.sum(ewT * wvec2[:, None], axis=0)
  e1_ref[...] = e1
  e2_ref[...] = e2
  c1 = jnp.mean(e1)
  c2 = jnp.mean(e2)
  m1 = jnp.maximum(jnp.max(e1), c1)
  m2 = jnp.maximum(jnp.max(e2), c2)
  g1 = _lrelu(jnp.max(ss1) + jnp.max(sd1) + m1)
  g1v_ref[...] = jnp.full((L,), g1, jnp.float32)
  c1_ref[0, 0] = c1
  c2_ref[0, 0] = c2
  m2_ref[0, 0] = m2


def _combine(acc_ref, den_ref, h_ref, ss_ref, sd_ref, cc, gg, b_ref):
  exl = jnp.exp(_lrelu(ss_ref[...] + sd_ref[...] + cc) - gg)
  den = (den_ref[0, :N] + den_ref[1, :N])[:, None] + exl + 1e-16
  num = acc_ref[0, :N] + acc_ref[1, :N] + exl * h_ref[...]
  return jax.nn.relu(num / den + b_ref[...][None, :])


def _mid_body(acc_ref, den_ref, h1_ref, ss1_ref, sd1_ref, c1_ref, g1_ref,
              m2_ref, b1_ref, w2_ref, as2_ref, ad2_ref,
              h2_ref, ss2_ref, sd2_ref, g2v_ref, g2_ref):
  x2 = _combine(acc_ref, den_ref, h1_ref, ss1_ref, sd1_ref,
                c1_ref[0, 0], g1_ref[0, 0], b1_ref)
  h2 = jnp.dot(x2, w2_ref[...], preferred_element_type=jnp.float32)
  h2_ref[...] = h2
  ss2 = jnp.sum(h2 * as2_ref[...][None, :], axis=1, keepdims=True)
  sd2 = jnp.sum(h2 * ad2_ref[...][None, :], axis=1, keepdims=True)
  ss2_ref[...] = ss2
  sd2_ref[...] = sd2
  g2 = _lrelu(jnp.max(ss2) + jnp.max(sd2) + m2_ref[0, 0])
  g2v_ref[...] = jnp.full((L,), g2, jnp.float32)
  g2_ref[0, 0] = g2


def _final_body(acc_ref, den_ref, h2_ref, ss2_ref, sd2_ref, c2_ref, g2_ref,
                b2_ref, xe_ref, wla_ref, wlb_ref, bl_ref, out_ref):
  x3 = _combine(acc_ref, den_ref, h2_ref, ss2_ref, sd2_ref,
                c2_ref[0, 0], g2_ref[0, 0], b2_ref)
  z = jnp.dot(x3, wla_ref[...], preferred_element_type=jnp.float32)
  z = z + jnp.dot(xe_ref[...], wlb_ref[...], preferred_element_type=jnp.float32)
  out_ref[...] = jax.nn.relu(z + bl_ref[...][None, :])


def _edge_body(src_h, dst_h, e_h, ss_h, sd_h, h_h, g_h,
               acc_o, den_o,
               acc_sh, den_sh, srcb, dstb, eb, rows_in,
               rows_out, ssg, sdg, exf, zbuf, gb, z1,
               gsem0, gsem1, ssem0, ssem1, dsem):
  gsem = (gsem0, gsem1)
  ssem = (ssem0, ssem1)
  c = lax.axis_index("c")
  s = lax.axis_index("s")

  pltpu.sync_copy(g_h, gb)
  gvec = plsc.load_gather(gb, [jnp.zeros((L,), jnp.int32)])

  zero16 = jnp.zeros((L,), jnp.float32)
  iota16 = lax.iota(jnp.int32, L)

  for i in range(ZR):
    for r in range(D // L):
      plsc.store_scatter(zbuf, [jnp.full((L,), i, jnp.int32), r * L + iota16],
                         zero16)
  for k in range(DEN_T // L):
    plsc.store_scatter(z1, [k * L + iota16], zero16)
  row0 = s * RPT

  @pl.loop(0, RPT // ZR)
  def _zero_acc(i):
    pltpu.sync_copy(zbuf, acc_sh.at[pl.ds(row0 + i * ZR, ZR)])

  pltpu.sync_copy(z1, den_sh.at[pl.ds(s * DEN_T, DEN_T)])

  plsc.subcore_barrier()

  w = c * NS + s

  def _start_fetch(j, b):
    pltpu.make_async_copy(h_h.at[srcb.at[j]], rows_in.at[b], gsem[b]).start()
    pltpu.make_async_copy(ss_h.at[srcb.at[j]], ssg.at[b], gsem[b]).start()
    pltpu.make_async_copy(sd_h.at[dstb.at[j]], sdg.at[b], gsem[b]).start()

  def _wait_fetch(j, b):
    pltpu.make_async_copy(h_h.at[srcb.at[j]], rows_in.at[b], gsem[b]).wait()
    pltpu.make_async_copy(ss_h.at[srcb.at[j]], ssg.at[b], gsem[b]).wait()
    pltpu.make_async_copy(sd_h.at[dstb.at[j]], sdg.at[b], gsem[b]).wait()

  @pl.loop(0, NBIG)
  def _big(g):
    pltpu.sync_copy(src_h.at[w, g], srcb)
    pltpu.sync_copy(dst_h.at[w, g], dstb)
    pltpu.sync_copy(e_h.at[w, g], eb)

    _start_fetch(0, 0)
    for j in range(CB):
      b = j & 1
      if j + 1 < CB:
        _start_fetch(j + 1, 1 - b)
      _wait_fetch(j, b)
      if j >= 2:
        pltpu.make_async_copy(rows_out.at[b], acc_sh.at[dstb.at[j - 2]],
                              ssem[b]).wait()

      for q in range(C // L):
        off = q * L
        ssv = ssg[b, pl.ds(off, L)]
        sdv = sdg[b, pl.ds(off, L)]
        ev = eb[j, pl.ds(off, L)]
        alpha = ssv + sdv + ev
        ex = jnp.exp(_lrelu(alpha) - gvec)
        exf[j, pl.ds(off, L)] = ex
        for i in range(L):
          coef = jnp.broadcast_to(ex[i], (L,))
          for r in range(D // L):
            rows_out[b, off + i, pl.ds(r * L, L)] = (
                rows_in[b, off + i, pl.ds(r * L, L)] * coef)

      pltpu.make_async_copy(rows_out.at[b], acc_sh.at[dstb.at[j]],
                            ssem[b]).start(add=True)
      pltpu.make_async_copy(exf.at[j], den_sh.at[dstb.at[j]],
                            dsem).start(add=True)

    for j in range(CB):
      pltpu.make_async_copy(exf.at[j], den_sh.at[dstb.at[j]], dsem).wait()
    pltpu.make_async_copy(rows_out.at[(CB - 2) & 1],
                          acc_sh.at[dstb.at[CB - 2]],
                          ssem[(CB - 2) & 1]).wait()
    pltpu.make_async_copy(rows_out.at[(CB - 1) & 1],
                          acc_sh.at[dstb.at[CB - 1]],
                          ssem[(CB - 1) & 1]).wait()

  plsc.subcore_barrier()

  pltpu.sync_copy(acc_sh.at[pl.ds(row0, RPT)], acc_o.at[c, pl.ds(row0, RPT)])
  pltpu.sync_copy(den_sh.at[pl.ds(s * DEN_T, DEN_T)],
                  den_o.at[c, pl.ds(s * DEN_T, DEN_T)])


_edge_pass = pl.kernel(
    _edge_body,
    out_type=[
        jax.ShapeDtypeStruct((NC, NP, D), jnp.float32),
        jax.ShapeDtypeStruct((NC, NS * DEN_T), jnp.float32),
    ],
    mesh=plsc.VectorSubcoreMesh(core_axis_name="c", subcore_axis_name="s",
                                num_cores=NC, num_subcores=NS),
    compiler_params=pltpu.CompilerParams(needs_layout_passes=False),
    scratch_types=[
        pltpu.VMEM_SHARED((NP, D), jnp.float32),
        pltpu.VMEM_SHARED((NS * DEN_T,), jnp.float32),
        pltpu.VMEM((CB, C), jnp.int32),
        pltpu.VMEM((CB, C), jnp.int32),
        pltpu.VMEM((CB, C), jnp.float32),
        pltpu.VMEM((2, C, D), jnp.float32),
        pltpu.VMEM((2, C, D), jnp.float32),
        pltpu.VMEM((2, C), jnp.float32),
        pltpu.VMEM((2, C), jnp.float32),
        pltpu.VMEM((CB, C), jnp.float32),
        pltpu.VMEM((ZR, D), jnp.float32),
        pltpu.VMEM((L,), jnp.float32),
        pltpu.VMEM((DEN_T,), jnp.float32),
        pltpu.SemaphoreType.DMA,
        pltpu.SemaphoreType.DMA,
        pltpu.SemaphoreType.DMA,
        pltpu.SemaphoreType.DMA,
        pltpu.SemaphoreType.DMA,
    ],
)


@jax.jit
def kernel(x, x_ext, edge_index, edge_weight, W1, att_src1, att_dst1, We1,
           att_e1, b1, W2, att_src2, att_dst2, We2, att_e2, b2, W_lin, b_lin):
  src = edge_index[0]
  dst = edge_index[1]
  ewT = edge_weight.T
  w1a = W1[:, :D].T
  w1b = W1[:, D:].T
  w2t = W2.T
  wla = W_lin[:, :H].T
  wlb = W_lin[:, H:].T

  f32 = jnp.float32
  prep = pl.pallas_call(
      _prep_body,
      out_shape=[
          jax.ShapeDtypeStruct((N, D), f32),
          jax.ShapeDtypeStruct((N, 1), f32),
          jax.ShapeDtypeStruct((N, 1), f32),
          jax.ShapeDtypeStruct((E,), f32),
          jax.ShapeDtypeStruct((E,), f32),
          jax.ShapeDtypeStruct((L,), f32),
          jax.ShapeDtypeStruct((1, 1), f32),
          jax.ShapeDtypeStruct((1, 1), f32),
          jax.ShapeDtypeStruct((1, 1), f32),
      ],
      in_specs=[_VMEM_SPEC] * 11,
      out_specs=[_VMEM_SPEC] * 6 + [_SMEM_SPEC] * 3,
  )
  h1, ss1, sd1, e1, e2, g1v, c1, c2, m2 = prep(
      x, x_ext, ewT, w1a, w1b, att_src1, att_dst1, We1, att_e1, We2, att_e2)

  eshape = (NC * NS, NBIG, CB, C)
  src2 = src.reshape(eshape)
  dst2 = dst.reshape(eshape)
  acc1, den1 = _edge_pass(src2, dst2, e1.reshape(eshape), ss1.reshape(N),
                          sd1.reshape(N), h1, g1v)

  g1s = g1v[:1].reshape(1, 1)
  mid = pl.pallas_call(
      _mid_body,
      out_shape=[
          jax.ShapeDtypeStruct((N, D), f32),
          jax.ShapeDtypeStruct((N, 1), f32),
          jax.ShapeDtypeStruct((N, 1), f32),
          jax.ShapeDtypeStruct((L,), f32),
          jax.ShapeDtypeStruct((1, 1), f32),
      ],
      in_specs=[_VMEM_SPEC] * 5 + [_SMEM_SPEC] * 3 + [_VMEM_SPEC] * 4,
      out_specs=[_VMEM_SPEC] * 4 + [_SMEM_SPEC],
  )
  h2, ss2, sd2, g2v, g2 = mid(acc1, den1, h1, ss1, sd1, c1, g1s, m2, b1, w2t,
                              att_src2, att_dst2)

  acc2, den2 = _edge_pass(src2, dst2, e2.reshape(eshape), ss2.reshape(N),
                          sd2.reshape(N), h2, g2v)

  fin = pl.pallas_call(
      _final_body,
      out_shape=jax.ShapeDtypeStruct((N, 2), f32),
      in_specs=[_VMEM_SPEC] * 5 + [_SMEM_SPEC] * 2 + [_VMEM_SPEC] * 5,
      out_specs=_VMEM_SPEC,
  )
  out = fin(acc2, den2, h2, ss2, sd2, c2, g2, b2, x_ext, wla, wlb, b_lin)
  return out


# speedup vs baseline: 30.8566x; 1.0641x over previous
import jax
import jax.numpy as jnp
from jax import lax
from jax.experimental import pallas as pl
from jax.experimental.pallas import tpu as pltpu
from jax.experimental.pallas import tpu_sc as plsc

N = 10000
E = 320000
D = 128
XE = 3
H = 128
ED = 4

NC = 2
NS = 16
L = 16

EPC = E // NC
EW = E // (NC * NS)
C = 80
NROW = E // C
RPW = EW // C
CB = 5
NBIG = RPW // CB
RPT = 632
NP = NS * RPT
ZR = 8
DEN_T = 640

_SLOPE = 0.2

_VMEM_SPEC = pl.BlockSpec(memory_space=pltpu.MemorySpace.VMEM)
_SMEM_SPEC = pl.BlockSpec(memory_space=pltpu.MemorySpace.SMEM)


def _lrelu(x):
  return jnp.where(x >= 0, x, _SLOPE * x)


def _prep_body(x_ref, xe_ref, ewT_ref, w1a_ref, w1b_ref, as1_ref, ad1_ref,
               we1_ref, ae1_ref, we2_ref, ae2_ref,
               h1_ref, ss1_ref, sd1_ref, e1_ref, e2_ref,
               g1v_ref, c1_ref, c2_ref, m2_ref):
  x = x_ref[...]
  xe = xe_ref[...]
  h1 = jnp.dot(x, w1a_ref[...], preferred_element_type=jnp.float32)
  h1 = h1 + jnp.dot(xe, w1b_ref[...], preferred_element_type=jnp.float32)
  h1_ref[...] = h1
  ss1 = jnp.sum(h1 * as1_ref[...][None, :], axis=1, keepdims=True)
  sd1 = jnp.sum(h1 * ad1_ref[...][None, :], axis=1, keepdims=True)
  ss1_ref[...] = ss1
  sd1_ref[...] = sd1
  wvec1 = jnp.sum(we1_ref[...] * ae1_ref[...][:, None], axis=0)
  wvec2 = jnp.sum(we2_ref[...] * ae2_ref[...][:, None], axis=0)
  ewT = ewT_ref[...]
  e1 = jnp.sum(ewT * wvec1[:, None], axis=0)
  e2 = jnp.sum(ewT * wvec2[:, None], axis=0)
  e1_ref[...] = e1
  e2_ref[...] = e2
  c1 = jnp.mean(e1)
  c2 = jnp.mean(e2)
  m1 = jnp.maximum(jnp.max(e1), c1)
  m2 = jnp.maximum(jnp.max(e2), c2)
  g1 = _lrelu(jnp.max(ss1) + jnp.max(sd1) + m1)
  g1v_ref[...] = jnp.full((L,), g1, jnp.float32)
  c1_ref[0, 0] = c1
  c2_ref[0, 0] = c2
  m2_ref[0, 0] = m2


def _combine(acc_ref, den_ref, h_ref, ss_ref, sd_ref, cc, gg, b_ref):
  exl = jnp.exp(_lrelu(ss_ref[...] + sd_ref[...] + cc) - gg)
  den = (den_ref[0, :N] + den_ref[1, :N])[:, None] + exl + 1e-16
  num = acc_ref[0, :N] + acc_ref[1, :N] + exl * h_ref[...]
  return jax.nn.relu(num / den + b_ref[...][None, :])


def _mid_body(acc_ref, den_ref, h1_ref, ss1_ref, sd1_ref, c1_ref, g1_ref,
              m2_ref, b1_ref, w2_ref, as2_ref, ad2_ref,
              h2_ref, ss2_ref, sd2_ref, g2v_ref, g2_ref):
  x2 = _combine(acc_ref, den_ref, h1_ref, ss1_ref, sd1_ref,
                c1_ref[0, 0], g1_ref[0, 0], b1_ref)
  h2 = jnp.dot(x2, w2_ref[...], preferred_element_type=jnp.float32)
  h2_ref[...] = h2
  ss2 = jnp.sum(h2 * as2_ref[...][None, :], axis=1, keepdims=True)
  sd2 = jnp.sum(h2 * ad2_ref[...][None, :], axis=1, keepdims=True)
  ss2_ref[...] = ss2
  sd2_ref[...] = sd2
  g2 = _lrelu(jnp.max(ss2) + jnp.max(sd2) + m2_ref[0, 0])
  g2v_ref[...] = jnp.full((L,), g2, jnp.float32)
  g2_ref[0, 0] = g2


def _final_body(acc_ref, den_ref, h2_ref, ss2_ref, sd2_ref, c2_ref, g2_ref,
                b2_ref, xe_ref, wla_ref, wlb_ref, bl_ref, out_ref):
  x3 = _combine(acc_ref, den_ref, h2_ref, ss2_ref, sd2_ref,
                c2_ref[0, 0], g2_ref[0, 0], b2_ref)
  z = jnp.dot(x3, wla_ref[...], preferred_element_type=jnp.float32)
  z = z + jnp.dot(xe_ref[...], wlb_ref[...], preferred_element_type=jnp.float32)
  out_ref[...] = jax.nn.relu(z + bl_ref[...][None, :])


def _edge_body(src_h, dst_h, e_h, ss_h, sd_h, h_h, g_h,
               acc_o, den_o,
               acc_sh, den_sh, srcb, dstb, eb, rows_in,
               rows_out, ssg, sdg, exf, zbuf, gb, z1,
               gsem0, gsem1, ssem0, ssem1, dsem):
  gsem = (gsem0, gsem1)
  ssem = (ssem0, ssem1)
  c = lax.axis_index("c")
  s = lax.axis_index("s")

  pltpu.sync_copy(g_h, gb)
  gvec = plsc.load_gather(gb, [jnp.zeros((L,), jnp.int32)])

  zero16 = jnp.zeros((L,), jnp.float32)
  iota16 = lax.iota(jnp.int32, L)

  for i in range(ZR):
    for r in range(D // L):
      plsc.store_scatter(zbuf, [jnp.full((L,), i, jnp.int32), r * L + iota16],
                         zero16)
  for k in range(DEN_T // L):
    plsc.store_scatter(z1, [k * L + iota16], zero16)
  row0 = s * RPT

  @pl.loop(0, RPT // ZR)
  def _zero_acc(i):
    pltpu.sync_copy(zbuf, acc_sh.at[pl.ds(row0 + i * ZR, ZR)])

  pltpu.sync_copy(z1, den_sh.at[pl.ds(s * DEN_T, DEN_T)])

  plsc.subcore_barrier()

  w = c * NS + s

  def _start_fetch(pp, j, b):
    pltpu.make_async_copy(h_h.at[srcb.at[j]], rows_in.at[b],
                          gsem[b]).start()
    pltpu.make_async_copy(ss_h.at[srcb.at[j]], ssg.at[b], gsem[b]).start()
    pltpu.make_async_copy(sd_h.at[dstb.at[pp, j]], sdg.at[b], gsem[b]).start()

  def _wait_fetch(pp, j, b):
    pltpu.make_async_copy(h_h.at[srcb.at[j]], rows_in.at[b],
                          gsem[b]).wait()
    pltpu.make_async_copy(ss_h.at[srcb.at[j]], ssg.at[b], gsem[b]).wait()
    pltpu.make_async_copy(sd_h.at[dstb.at[pp, j]], sdg.at[b], gsem[b]).wait()

  def _wait_row_scatter(bb):
    pltpu.make_async_copy(rows_out.at[bb], acc_sh.at[dstb.at[0, 0]],
                          ssem[bb]).wait()

  def _wait_den():
    pltpu.make_async_copy(exf.at[pl.ds(0, C)], den_sh.at[dstb.at[0, 0]],
                          dsem).wait()

  @pl.loop(0, NBIG)
  def _big(g):
    p = g % 2
    pltpu.sync_copy((src_h.at[w, g], dst_h.at[w, g], e_h.at[w, g]),
                    (srcb, dstb.at[p], eb))

    _start_fetch(p, 0, 0)
    for j in range(CB):
      b = j & 1
      if j + 1 < CB:
        _start_fetch(p, j + 1, 1 - b)
      _wait_fetch(p, j, b)
      if j >= 2:
        _wait_row_scatter(b)
      else:
        @pl.when(g >= 1)
        def _cross_chunk_drain():
          _wait_row_scatter(b)

      for q in range(C // L):
        off = q * L
        ssv = ssg[b, pl.ds(off, L)]
        sdv = sdg[b, pl.ds(off, L)]
        ev = eb[j, pl.ds(off, L)]
        alpha = ssv + sdv + ev
        ex = jnp.exp(_lrelu(alpha) - gvec)
        plsc.store_scatter(exf, [jnp.full((L,), p * (CB * C) + j * C + off,
                                          jnp.int32) + iota16], ex)
        for i in range(L):
          coef = jnp.broadcast_to(ex[i], (L,))
          for r in range(D // L):
            rows_out[b, off + i, pl.ds(r * L, L)] = (
                rows_in[b, off + i, pl.ds(r * L, L)] * coef)

      pltpu.make_async_copy(rows_out.at[b], acc_sh.at[dstb.at[p, j]],
                            ssem[b]).start(add=True)

    @pl.when(g >= 1)
    def _drain_prev_den():
      for _ in range(CB):
        _wait_den()
    for j in range(CB):
      pltpu.make_async_copy(exf.at[pl.ds(p * (CB * C) + j * C, C)],
                            den_sh.at[dstb.at[p, j]],
                            dsem).start(add=True)

  for _ in range(CB):
    _wait_den()
  _wait_row_scatter((CB - 2) & 1)
  _wait_row_scatter((CB - 1) & 1)

  plsc.subcore_barrier()

  pltpu.sync_copy(acc_sh.at[pl.ds(row0, RPT)], acc_o.at[c, pl.ds(row0, RPT)])
  pltpu.sync_copy(den_sh.at[pl.ds(s * DEN_T, DEN_T)],
                  den_o.at[c, pl.ds(s * DEN_T, DEN_T)])


_edge_pass = pl.kernel(
    _edge_body,
    out_type=[
        jax.ShapeDtypeStruct((NC, NP, D), jnp.float32),
        jax.ShapeDtypeStruct((NC, NS * DEN_T), jnp.float32),
    ],
    mesh=plsc.VectorSubcoreMesh(core_axis_name="c", subcore_axis_name="s",
                                num_cores=NC, num_subcores=NS),
    compiler_params=pltpu.CompilerParams(needs_layout_passes=False),
    scratch_types=[
        pltpu.VMEM_SHARED((NP, D), jnp.float32),
        pltpu.VMEM_SHARED((NS * DEN_T,), jnp.float32),
        pltpu.VMEM((CB, C), jnp.int32),
        pltpu.VMEM((2, CB, C), jnp.int32),
        pltpu.VMEM((CB, C), jnp.float32),
        pltpu.VMEM((2, C, D), jnp.float32),
        pltpu.VMEM((2, C, D), jnp.float32),
        pltpu.VMEM((2, C), jnp.float32),
        pltpu.VMEM((2, C), jnp.float32),
        pltpu.VMEM((2 * CB * C,), jnp.float32),
        pltpu.VMEM((ZR, D), jnp.float32),
        pltpu.VMEM((L,), jnp.float32),
        pltpu.VMEM((DEN_T,), jnp.float32),
        pltpu.SemaphoreType.DMA,
        pltpu.SemaphoreType.DMA,
        pltpu.SemaphoreType.DMA,
        pltpu.SemaphoreType.DMA,
        pltpu.SemaphoreType.DMA,
    ],
)


@jax.jit
def kernel(x, x_ext, edge_index, edge_weight, W1, att_src1, att_dst1, We1,
           att_e1, b1, W2, att_src2, att_dst2, We2, att_e2, b2, W_lin, b_lin):
  src = edge_index[0]
  dst = edge_index[1]
  ewT = edge_weight.T
  w1a = W1[:, :D].T
  w1b = W1[:, D:].T
  w2t = W2.T
  wla = W_lin[:, :H].T
  wlb = W_lin[:, H:].T

  f32 = jnp.float32
  prep = pl.pallas_call(
      _prep_body,
      out_shape=[
          jax.ShapeDtypeStruct((N, D), f32),
          jax.ShapeDtypeStruct((N, 1), f32),
          jax.ShapeDtypeStruct((N, 1), f32),
          jax.ShapeDtypeStruct((E,), f32),
          jax.ShapeDtypeStruct((E,), f32),
          jax.ShapeDtypeStruct((L,), f32),
          jax.ShapeDtypeStruct((1, 1), f32),
          jax.ShapeDtypeStruct((1, 1), f32),
          jax.ShapeDtypeStruct((1, 1), f32),
      ],
      in_specs=[_VMEM_SPEC] * 11,
      out_specs=[_VMEM_SPEC] * 6 + [_SMEM_SPEC] * 3,
  )
  h1, ss1, sd1, e1, e2, g1v, c1, c2, m2 = prep(
      x, x_ext, ewT, w1a, w1b, att_src1, att_dst1, We1, att_e1, We2, att_e2)

  eshape = (NC * NS, NBIG, CB, C)
  src2 = src.reshape(eshape)
  dst2 = dst.reshape(eshape)
  acc1, den1 = _edge_pass(src2, dst2, e1.reshape(eshape), ss1.reshape(N),
                          sd1.reshape(N), h1, g1v)

  g1s = g1v[:1].reshape(1, 1)
  mid = pl.pallas_call(
      _mid_body,
      out_shape=[
          jax.ShapeDtypeStruct((N, D), f32),
          jax.ShapeDtypeStruct((N, 1), f32),
          jax.ShapeDtypeStruct((N, 1), f32),
          jax.ShapeDtypeStruct((L,), f32),
          jax.ShapeDtypeStruct((1, 1), f32),
      ],
      in_specs=[_VMEM_SPEC] * 5 + [_SMEM_SPEC] * 3 + [_VMEM_SPEC] * 4,
      out_specs=[_VMEM_SPEC] * 4 + [_SMEM_SPEC],
  )
  h2, ss2, sd2, g2v, g2 = mid(acc1, den1, h1, ss1, sd1, c1, g1s, m2, b1, w2t,
                              att_src2, att_dst2)

  acc2, den2 = _edge_pass(src2, dst2, e2.reshape(eshape), ss2.reshape(N),
                          sd2.reshape(N), h2, g2v)

  fin = pl.pallas_call(
      _final_body,
      out_shape=jax.ShapeDtypeStruct((N, 2), f32),
      in_specs=[_VMEM_SPEC] * 5 + [_SMEM_SPEC] * 2 + [_VMEM_SPEC] * 5,
      out_specs=_VMEM_SPEC,
  )
  out = fin(acc2, den2, h2, ss2, sd2, c2, g2, b2, x_ext, wla, wlb, b_lin)
  return out


# speedup vs baseline: 30.9834x; 1.0041x over previous
import jax
import jax.numpy as jnp
from jax import lax
from jax.experimental import pallas as pl
from jax.experimental.pallas import tpu as pltpu
from jax.experimental.pallas import tpu_sc as plsc

N = 10000
E = 320000
D = 128
XE = 3
H = 128
ED = 4

NC = 2
NS = 16
L = 16

EPC = E // NC
EW = E // (NC * NS)
C = 80
NROW = E // C
RPW = EW // C
CB = 5
NBIG = RPW // CB
RPT = 632
NP = NS * RPT
ZR = 16
DEN_T = 640

_SLOPE = 0.2

_VMEM_SPEC = pl.BlockSpec(memory_space=pltpu.MemorySpace.VMEM)
_SMEM_SPEC = pl.BlockSpec(memory_space=pltpu.MemorySpace.SMEM)


def _lrelu(x):
  return jnp.where(x >= 0, x, _SLOPE * x)


def _prep_body(x_ref, xe_ref, ewT_ref, w1a_ref, w1b_ref, as1_ref, ad1_ref,
               we1_ref, ae1_ref, we2_ref, ae2_ref,
               h1_ref, ss1_ref, sd1_ref, e1_ref, e2_ref,
               g1v_ref, c1_ref, c2_ref, m2_ref):
  x = x_ref[...]
  xe = xe_ref[...]
  h1 = jnp.dot(x, w1a_ref[...], preferred_element_type=jnp.float32)
  h1 = h1 + jnp.dot(xe, w1b_ref[...], preferred_element_type=jnp.float32)
  h1_ref[...] = h1
  ss1 = jnp.sum(h1 * as1_ref[...][None, :], axis=1, keepdims=True)
  sd1 = jnp.sum(h1 * ad1_ref[...][None, :], axis=1, keepdims=True)
  ss1_ref[...] = ss1
  sd1_ref[...] = sd1
  wvec1 = jnp.sum(we1_ref[...] * ae1_ref[...][:, None], axis=0)
  wvec2 = jnp.sum(we2_ref[...] * ae2_ref[...][:, None], axis=0)
  ewT = ewT_ref[...]
  e1 = jnp.sum(ewT * wvec1[:, None], axis=0)
  e2 = jnp.sum(ewT * wvec2[:, None], axis=0)
  e1_ref[...] = e1
  e2_ref[...] = e2
  c1 = jnp.mean(e1)
  c2 = jnp.mean(e2)
  m1 = jnp.maximum(jnp.max(e1), c1)
  m2 = jnp.maximum(jnp.max(e2), c2)
  g1 = _lrelu(jnp.max(ss1) + jnp.max(sd1) + m1)
  g1v_ref[...] = jnp.full((L,), g1, jnp.float32)
  c1_ref[0, 0] = c1
  c2_ref[0, 0] = c2
  m2_ref[0, 0] = m2


def _combine(acc_ref, den_ref, h_ref, ss_ref, sd_ref, cc, gg, b_ref):
  exl = jnp.exp(_lrelu(ss_ref[...] + sd_ref[...] + cc) - gg)
  den = (den_ref[0, :N] + den_ref[1, :N])[:, None] + exl + 1e-16
  num = acc_ref[0, :N] + acc_ref[1, :N] + exl * h_ref[...]
  return jax.nn.relu(num / den + b_ref[...][None, :])


def _mid_body(acc_ref, den_ref, h1_ref, ss1_ref, sd1_ref, c1_ref, g1_ref,
              m2_ref, b1_ref, w2_ref, as2_ref, ad2_ref,
              h2_ref, ss2_ref, sd2_ref, g2v_ref, g2_ref):
  x2 = _combine(acc_ref, den_ref, h1_ref, ss1_ref, sd1_ref,
                c1_ref[0, 0], g1_ref[0, 0], b1_ref)
  h2 = jnp.dot(x2, w2_ref[...], preferred_element_type=jnp.float32)
  h2_ref[...] = h2
  ss2 = jnp.sum(h2 * as2_ref[...][None, :], axis=1, keepdims=True)
  sd2 = jnp.sum(h2 * ad2_ref[...][None, :], axis=1, keepdims=True)
  ss2_ref[...] = ss2
  sd2_ref[...] = sd2
  g2 = _lrelu(jnp.max(ss2) + jnp.max(sd2) + m2_ref[0, 0])
  g2v_ref[...] = jnp.full((L,), g2, jnp.float32)
  g2_ref[0, 0] = g2


def _final_body(acc_ref, den_ref, h2_ref, ss2_ref, sd2_ref, c2_ref, g2_ref,
                b2_ref, xe_ref, wla_ref, wlb_ref, bl_ref, out_ref):
  x3 = _combine(acc_ref, den_ref, h2_ref, ss2_ref, sd2_ref,
                c2_ref[0, 0], g2_ref[0, 0], b2_ref)
  z = jnp.dot(x3, wla_ref[...], preferred_element_type=jnp.float32)
  z = z + jnp.dot(xe_ref[...], wlb_ref[...], preferred_element_type=jnp.float32)
  out_ref[...] = jax.nn.relu(z + bl_ref[...][None, :])


def _edge_body(src_h, dst_h, e_h, ss_h, sd_h, h_h, g_h,
               acc_o, den_o,
               acc_sh, den_sh, srcb, dstb, eb, rows_in,
               rows_out, ssg, sdg, exf, zbuf, gb, z1,
               gsem0, gsem1, ssem0, ssem1, dsem):
  gsem = (gsem0, gsem1)
  ssem = (ssem0, ssem1)
  c = lax.axis_index("c")
  s = lax.axis_index("s")

  pltpu.sync_copy(g_h, gb)
  gvec = plsc.load_gather(gb, [jnp.zeros((L,), jnp.int32)])

  zero16 = jnp.zeros((L,), jnp.float32)
  iota16 = lax.iota(jnp.int32, L)

  for i in range(ZR):
    for r in range(D // L):
      plsc.store_scatter(zbuf, [jnp.full((L,), i, jnp.int32), r * L + iota16],
                         zero16)
  for k in range(DEN_T // L):
    plsc.store_scatter(z1, [k * L + iota16], zero16)
  row0 = s * RPT

  @pl.loop(0, RPT // ZR)
  def _zero_acc(i):
    pltpu.sync_copy(zbuf, acc_sh.at[pl.ds(row0 + i * ZR, ZR)])

  pltpu.sync_copy(zbuf.at[pl.ds(0, RPT % ZR)],
                  acc_sh.at[pl.ds(row0 + (RPT // ZR) * ZR, RPT % ZR)])
  pltpu.sync_copy(z1, den_sh.at[pl.ds(s * DEN_T, DEN_T)])

  plsc.subcore_barrier()

  w = c * NS + s

  def _start_fetch(pp, j, b):
    pltpu.make_async_copy(h_h.at[srcb.at[j]], rows_in.at[b],
                          gsem[b]).start()
    pltpu.make_async_copy(ss_h.at[srcb.at[j]], ssg.at[b], gsem[b]).start()
    pltpu.make_async_copy(sd_h.at[dstb.at[pp, j]], sdg.at[b], gsem[b]).start()

  def _wait_fetch(pp, j, b):
    pltpu.make_async_copy(h_h.at[srcb.at[j]], rows_in.at[b],
                          gsem[b]).wait()
    pltpu.make_async_copy(ss_h.at[srcb.at[j]], ssg.at[b], gsem[b]).wait()
    pltpu.make_async_copy(sd_h.at[dstb.at[pp, j]], sdg.at[b], gsem[b]).wait()

  def _wait_row_scatter(bb):
    pltpu.make_async_copy(rows_out.at[bb], acc_sh.at[dstb.at[0, 0]],
                          ssem[bb]).wait()

  def _wait_den():
    pltpu.make_async_copy(exf.at[pl.ds(0, C)], den_sh.at[dstb.at[0, 0]],
                          dsem).wait()

  @pl.loop(0, NBIG)
  def _big(g):
    p = g % 2
    pltpu.sync_copy((src_h.at[w, g], dst_h.at[w, g], e_h.at[w, g]),
                    (srcb, dstb.at[p], eb))

    _start_fetch(p, 0, 0)
    for j in range(CB):
      b = j & 1
      if j + 1 < CB:
        _start_fetch(p, j + 1, 1 - b)
      _wait_fetch(p, j, b)
      if j >= 2:
        _wait_row_scatter(b)
      else:
        @pl.when(g >= 1)
        def _cross_chunk_drain():
          _wait_row_scatter(b)

      for q in range(C // L):
        off = q * L
        ssv = ssg[b, pl.ds(off, L)]
        sdv = sdg[b, pl.ds(off, L)]
        ev = eb[j, pl.ds(off, L)]
        alpha = ssv + sdv + ev
        ex = jnp.exp(_lrelu(alpha) - gvec)
        plsc.store_scatter(exf, [jnp.full((L,), p * (CB * C) + j * C + off,
                                          jnp.int32) + iota16], ex)
        for i in range(L):
          coef = jnp.broadcast_to(ex[i], (L,))
          for r in range(D // L):
            rows_out[b, off + i, pl.ds(r * L, L)] = (
                rows_in[b, off + i, pl.ds(r * L, L)] * coef)

      pltpu.make_async_copy(rows_out.at[b], acc_sh.at[dstb.at[p, j]],
                            ssem[b]).start(add=True)

    @pl.when(g >= 1)
    def _drain_prev_den():
      for _ in range(CB):
        _wait_den()
    for j in range(CB):
      pltpu.make_async_copy(exf.at[pl.ds(p * (CB * C) + j * C, C)],
                            den_sh.at[dstb.at[p, j]],
                            dsem).start(add=True)

  for _ in range(CB):
    _wait_den()
  _wait_row_scatter((CB - 2) & 1)
  _wait_row_scatter((CB - 1) & 1)

  plsc.subcore_barrier()

  pltpu.sync_copy(acc_sh.at[pl.ds(row0, RPT)], acc_o.at[c, pl.ds(row0, RPT)])
  pltpu.sync_copy(den_sh.at[pl.ds(s * DEN_T, DEN_T)],
                  den_o.at[c, pl.ds(s * DEN_T, DEN_T)])


_edge_pass = pl.kernel(
    _edge_body,
    out_type=[
        jax.ShapeDtypeStruct((NC, NP, D), jnp.float32),
        jax.ShapeDtypeStruct((NC, NS * DEN_T), jnp.float32),
    ],
    mesh=plsc.VectorSubcoreMesh(core_axis_name="c", subcore_axis_name="s",
                                num_cores=NC, num_subcores=NS),
    compiler_params=pltpu.CompilerParams(needs_layout_passes=False),
    scratch_types=[
        pltpu.VMEM_SHARED((NP, D), jnp.float32),
        pltpu.VMEM_SHARED((NS * DEN_T,), jnp.float32),
        pltpu.VMEM((CB, C), jnp.int32),
        pltpu.VMEM((2, CB, C), jnp.int32),
        pltpu.VMEM((CB, C), jnp.float32),
        pltpu.VMEM((2, C, D), jnp.float32),
        pltpu.VMEM((2, C, D), jnp.float32),
        pltpu.VMEM((2, C), jnp.float32),
        pltpu.VMEM((2, C), jnp.float32),
        pltpu.VMEM((2 * CB * C,), jnp.float32),
        pltpu.VMEM((ZR, D), jnp.float32),
        pltpu.VMEM((L,), jnp.float32),
        pltpu.VMEM((DEN_T,), jnp.float32),
        pltpu.SemaphoreType.DMA,
        pltpu.SemaphoreType.DMA,
        pltpu.SemaphoreType.DMA,
        pltpu.SemaphoreType.DMA,
        pltpu.SemaphoreType.DMA,
    ],
)


@jax.jit
def kernel(x, x_ext, edge_index, edge_weight, W1, att_src1, att_dst1, We1,
           att_e1, b1, W2, att_src2, att_dst2, We2, att_e2, b2, W_lin, b_lin):
  src = edge_index[0]
  dst = edge_index[1]
  ewT = edge_weight.T
  w1a = W1[:, :D].T
  w1b = W1[:, D:].T
  w2t = W2.T
  wla = W_lin[:, :H].T
  wlb = W_lin[:, H:].T

  f32 = jnp.float32
  prep = pl.pallas_call(
      _prep_body,
      out_shape=[
          jax.ShapeDtypeStruct((N, D), f32),
          jax.ShapeDtypeStruct((N, 1), f32),
          jax.ShapeDtypeStruct((N, 1), f32),
          jax.ShapeDtypeStruct((E,), f32),
          jax.ShapeDtypeStruct((E,), f32),
          jax.ShapeDtypeStruct((L,), f32),
          jax.ShapeDtypeStruct((1, 1), f32),
          jax.ShapeDtypeStruct((1, 1), f32),
          jax.ShapeDtypeStruct((1, 1), f32),
      ],
      in_specs=[_VMEM_SPEC] * 11,
      out_specs=[_VMEM_SPEC] * 6 + [_SMEM_SPEC] * 3,
  )
  h1, ss1, sd1, e1, e2, g1v, c1, c2, m2 = prep(
      x, x_ext, ewT, w1a, w1b, att_src1, att_dst1, We1, att_e1, We2, att_e2)

  eshape = (NC * NS, NBIG, CB, C)
  src2 = src.reshape(eshape)
  dst2 = dst.reshape(eshape)
  acc1, den1 = _edge_pass(src2, dst2, e1.reshape(eshape), ss1.reshape(N),
                          sd1.reshape(N), h1, g1v)

  g1s = g1v[:1].reshape(1, 1)
  mid = pl.pallas_call(
      _mid_body,
      out_shape=[
          jax.ShapeDtypeStruct((N, D), f32),
          jax.ShapeDtypeStruct((N, 1), f32),
          jax.ShapeDtypeStruct((N, 1), f32),
          jax.ShapeDtypeStruct((L,), f32),
          jax.ShapeDtypeStruct((1, 1), f32),
      ],
      in_specs=[_VMEM_SPEC] * 5 + [_SMEM_SPEC] * 3 + [_VMEM_SPEC] * 4,
      out_specs=[_VMEM_SPEC] * 4 + [_SMEM_SPEC],
  )
  h2, ss2, sd2, g2v, g2 = mid(acc1, den1, h1, ss1, sd1, c1, g1s, m2, b1, w2t,
                              att_src2, att_dst2)

  acc2, den2 = _edge_pass(src2, dst2, e2.reshape(eshape), ss2.reshape(N),
                          sd2.reshape(N), h2, g2v)

  fin = pl.pallas_call(
      _final_body,
      out_shape=jax.ShapeDtypeStruct((N, 2), f32),
      in_specs=[_VMEM_SPEC] * 5 + [_SMEM_SPEC] * 2 + [_VMEM_SPEC] * 5,
      out_specs=_VMEM_SPEC,
  )
  out = fin(acc2, den2, h2, ss2, sd2, c2, g2, b2, x_ext, wla, wlb, b_lin)
  return out
